# baseline (speedup 1.0000x reference)
import jax
import jax.numpy as jnp
from jax.experimental import pallas as pl
from jax.experimental.pallas import tpu as pltpu

_EPS = 1e-5
_VMEM = 64 * 1024 * 1024
_B = 4


def _edge_masks(L, W, dtype):
    col = jax.lax.broadcasted_iota(jnp.int32, (1, L), 1) % W
    left = (col != 0).astype(dtype)
    right = (col != W - 1).astype(dtype)
    return left, right


def _fill_shifted(buf, x, C, L, W, B):
    zero = jnp.zeros((3 * C, 2 * W), jnp.bfloat16)
    buf[:, 0:W + 1] = zero[:, :W + 1]
    for s in range(1, B):
        buf[:, pl.ds(s * L - W + 1, 2 * W)] = zero
    buf[:, pl.ds(B * L - W + 1, W + 1)] = zero[:, :W + 1]
    lo = {0: W + 1, 1: 1, 2: 0}
    hi = {0: L + 2, 1: L + 1, 2: L - W + 1}
    for b in range(B):
        for ky in range(3):
            i0 = max(lo[ky], 0 if b == 0 else 1)
            i1 = min(hi[ky], L + 2 if b == B - 1 else L + 1)
            s0 = i0 - 1 + (ky - 1) * W
            buf[ky * C:(ky + 1) * C, pl.ds(b * L + i0, i1 - i0)] = (
                x[b, :, s0:s0 + i1 - i0])


def _conv_taps(buf, w_ref, L, W):
    ml, mr = _edge_masks(L, W, jnp.bfloat16)
    acc = jnp.dot(w_ref[0], buf[:, pl.ds(0, L)] * ml,
                  preferred_element_type=jnp.float32)
    acc = acc + jnp.dot(w_ref[1], buf[:, pl.ds(1, L)],
                        preferred_element_type=jnp.float32)
    acc = acc + jnp.dot(w_ref[2], buf[:, pl.ds(2, L)] * mr,
                        preferred_element_type=jnp.float32)
    return acc


def _conv_stats_call(xf, w, *, N, H, W, Cin, Cout):
    L = H * W

    def body(x_ref, w_ref, o_ref, s1_ref, s2_ref, buf):
        _fill_shifted(buf, x_ref[...], Cin, L, W, _B)
        acc = _conv_taps(buf, w_ref, _B * L, W)
        s1_ref[0] = jnp.sum(acc, axis=1, keepdims=True)
        s2_ref[0] = jnp.sum(acc * acc, axis=1, keepdims=True)
        for b in range(_B):
            o_ref[b] = acc[:, b * L:(b + 1) * L].astype(jnp.bfloat16)

    return pl.pallas_call(
        body,
        out_shape=(
            jax.ShapeDtypeStruct((N, Cout, L), jnp.bfloat16),
            jax.ShapeDtypeStruct((N // _B, Cout, 1), jnp.float32),
            jax.ShapeDtypeStruct((N // _B, Cout, 1), jnp.float32),
        ),
        grid=(N // _B,),
        in_specs=[
            pl.BlockSpec((_B, Cin, L), lambda n: (n, 0, 0)),
            pl.BlockSpec((3, Cout, 3 * Cin), lambda n: (0, 0, 0)),
        ],
        out_specs=(
            pl.BlockSpec((_B, Cout, L), lambda n: (n, 0, 0)),
            pl.BlockSpec((1, Cout, 1), lambda n: (n, 0, 0)),
            pl.BlockSpec((1, Cout, 1), lambda n: (n, 0, 0)),
        ),
        scratch_shapes=[pltpu.VMEM((3 * Cin, _B * L + 2), jnp.bfloat16)],
        compiler_params=pltpu.CompilerParams(
            dimension_semantics=("parallel",),
            vmem_limit_bytes=_VMEM,
        ),
    )(xf, w)


def _finalize(s1_ref, s2_ref, g_ref, b_ref, m):
    mean = jnp.sum(s1_ref[...], axis=0) / m
    var = jnp.maximum(jnp.sum(s2_ref[...], axis=0) / m - mean * mean, 0.0)
    scale = g_ref[...] * jax.lax.rsqrt(var + _EPS)
    shift = b_ref[...] - mean * scale
    return scale, shift


def _bn_conv_stats_call(c1, w, s1, s2, g, b, *, N, H, W, Cin, Cout):
    L = H * W
    m = float(N * L)

    def body(x_ref, w_ref, s1_ref, s2_ref, g_ref, b_ref,
             o_ref, t1_ref, t2_ref, buf):
        scale, shift = _finalize(s1_ref, s2_ref, g_ref, b_ref, m)
        y = jnp.maximum(x_ref[...].astype(jnp.float32) * scale + shift, 0.0)
        _fill_shifted(buf, y.astype(jnp.bfloat16), Cin, L, W, _B)
        acc = _conv_taps(buf, w_ref, _B * L, W)
        t1_ref[0] = jnp.sum(acc, axis=1, keepdims=True)
        t2_ref[0] = jnp.sum(acc * acc, axis=1, keepdims=True)
        for b in range(_B):
            o_ref[b] = acc[:, b * L:(b + 1) * L].astype(jnp.bfloat16)

    return pl.pallas_call(
        body,
        out_shape=(
            jax.ShapeDtypeStruct((N, Cout, L), jnp.bfloat16),
            jax.ShapeDtypeStruct((N // _B, Cout, 1), jnp.float32),
            jax.ShapeDtypeStruct((N // _B, Cout, 1), jnp.float32),
        ),
        grid=(N // _B,),
        in_specs=[
            pl.BlockSpec((_B, Cin, L), lambda n: (n, 0, 0)),
            pl.BlockSpec((3, Cout, 3 * Cin), lambda n: (0, 0, 0)),
            pl.BlockSpec((N // _B, Cin, 1), lambda n: (0, 0, 0)),
            pl.BlockSpec((N // _B, Cin, 1), lambda n: (0, 0, 0)),
            pl.BlockSpec((Cin, 1), lambda n: (0, 0)),
            pl.BlockSpec((Cin, 1), lambda n: (0, 0)),
        ],
        out_specs=(
            pl.BlockSpec((_B, Cout, L), lambda n: (n, 0, 0)),
            pl.BlockSpec((1, Cout, 1), lambda n: (n, 0, 0)),
            pl.BlockSpec((1, Cout, 1), lambda n: (n, 0, 0)),
        ),
        scratch_shapes=[pltpu.VMEM((3 * Cin, _B * L + 2), jnp.bfloat16)],
        compiler_params=pltpu.CompilerParams(
            dimension_semantics=("parallel",),
            vmem_limit_bytes=_VMEM,
        ),
    )(c1, w, s1, s2, g, b)


def _bn_relu_out_call(c2, s1, s2, g, b, *, N, H, W, Cout):
    L = H * W
    m = float(N * L)

    def body(x_ref, s1_ref, s2_ref, g_ref, b_ref, o_ref):
        scale, shift = _finalize(s1_ref, s2_ref, g_ref, b_ref, m)
        o_ref[...] = jnp.maximum(
            x_ref[...].astype(jnp.float32) * scale + shift, 0.0)

    return pl.pallas_call(
        body,
        out_shape=jax.ShapeDtypeStruct((N, Cout, L), jnp.float32),
        grid=(N // _B,),
        in_specs=[
            pl.BlockSpec((_B, Cout, L), lambda n: (n, 0, 0)),
            pl.BlockSpec((N // _B, Cout, 1), lambda n: (0, 0, 0)),
            pl.BlockSpec((N // _B, Cout, 1), lambda n: (0, 0, 0)),
            pl.BlockSpec((Cout, 1), lambda n: (0, 0)),
            pl.BlockSpec((Cout, 1), lambda n: (0, 0)),
        ],
        out_specs=pl.BlockSpec((_B, Cout, L), lambda n: (n, 0, 0)),
        compiler_params=pltpu.CompilerParams(
            dimension_semantics=("parallel",),
            vmem_limit_bytes=_VMEM,
        ),
    )(c2, s1, s2, g, b)


def _pack_w(w, Cin, Cout):
    return (w.reshape(3, 3, Cin, Cout)
            .transpose(1, 3, 0, 2)
            .reshape(3, Cout, 3 * Cin)
            .astype(jnp.bfloat16))


def kernel(x, w1, g1, b1, w2, g2, b2):
    N, C0, H0, W0 = x.shape
    H, W = H0 // 2, W0 // 2
    L = H * W
    C1 = w1.shape[2]
    C2 = w2.shape[2]

    pooled = jnp.maximum(
        jnp.maximum(x[:, :, 0::2, 0::2], x[:, :, 0::2, 1::2]),
        jnp.maximum(x[:, :, 1::2, 0::2], x[:, :, 1::2, 1::2]))
    xf = pooled.astype(jnp.bfloat16).reshape(N, C0, L)

    w1p = _pack_w(w1, C0, C1)
    w2p = _pack_w(w2, C1, C2)
    g1r, b1r = g1.reshape(C1, 1), b1.reshape(C1, 1)
    g2r, b2r = g2.reshape(C2, 1), b2.reshape(C2, 1)

    c1, s1, s2 = _conv_stats_call(xf, w1p, N=N, H=H, W=W, Cin=C0, Cout=C1)
    c2, t1, t2 = _bn_conv_stats_call(c1, w2p, s1, s2, g1r, b1r,
                                     N=N, H=H, W=W, Cin=C1, Cout=C2)
    y = _bn_relu_out_call(c2, t1, t2, g2r, b2r, N=N, H=H, W=W, Cout=C2)
    return y.reshape(N, C2, H, W)


# speedup vs baseline: 9.5333x; 9.5333x over previous
import jax
import jax.numpy as jnp
from jax.experimental import pallas as pl
from jax.experimental.pallas import tpu as pltpu

_EPS = 1e-5
_VMEM = 64 * 1024 * 1024
_B = 4


def _edge_masks(L, W, dtype):
    col = jax.lax.broadcasted_iota(jnp.int32, (1, L), 1) % W
    left = (col != 0).astype(dtype)
    right = (col != W - 1).astype(dtype)
    return left, right


def _fill_shifted(buf, x, C, L, W, B):
    zero = jnp.zeros((3 * C, 2 * W), jnp.bfloat16)
    buf[:, 0:W + 1] = zero[:, :W + 1]
    for s in range(1, B):
        buf[:, pl.ds(s * L - W + 1, 2 * W)] = zero
    buf[:, pl.ds(B * L - W + 1, W + 1)] = zero[:, :W + 1]
    lo = {0: W + 1, 1: 1, 2: 0}
    hi = {0: L + 2, 1: L + 1, 2: L - W + 1}
    for b in range(B):
        for ky in range(3):
            i0 = max(lo[ky], 0 if b == 0 else 1)
            i1 = min(hi[ky], L + 2 if b == B - 1 else L + 1)
            s0 = i0 - 1 + (ky - 1) * W
            buf[ky * C:(ky + 1) * C, pl.ds(b * L + i0, i1 - i0)] = (
                x[b, :, s0:s0 + i1 - i0])


def _conv_taps(buf, w_ref, L, W):
    ml, mr = _edge_masks(L, W, jnp.bfloat16)
    acc = jnp.dot(w_ref[0], buf[:, pl.ds(0, L)] * ml,
                  preferred_element_type=jnp.float32)
    acc = acc + jnp.dot(w_ref[1], buf[:, pl.ds(1, L)],
                        preferred_element_type=jnp.float32)
    acc = acc + jnp.dot(w_ref[2], buf[:, pl.ds(2, L)] * mr,
                        preferred_element_type=jnp.float32)
    return acc


def _conv_stats_call(xf, w, *, N, H, W, Cin, Cout):
    L = H * W

    def body(x_ref, w_ref, o_ref, s1_ref, s2_ref, buf):
        _fill_shifted(buf, x_ref[...], Cin, L, W, _B)
        acc = _conv_taps(buf, w_ref, _B * L, W)
        s1_ref[0] = jnp.sum(acc, axis=1, keepdims=True)
        s2_ref[0] = jnp.sum(acc * acc, axis=1, keepdims=True)
        for b in range(_B):
            o_ref[b] = acc[:, b * L:(b + 1) * L].astype(jnp.bfloat16)

    return pl.pallas_call(
        body,
        out_shape=(
            jax.ShapeDtypeStruct((N, Cout, L), jnp.bfloat16),
            jax.ShapeDtypeStruct((N // _B, Cout, 1), jnp.float32),
            jax.ShapeDtypeStruct((N // _B, Cout, 1), jnp.float32),
        ),
        grid=(N // _B,),
        in_specs=[
            pl.BlockSpec((_B, Cin, L), lambda n: (n, 0, 0)),
            pl.BlockSpec((3, Cout, 3 * Cin), lambda n: (0, 0, 0)),
        ],
        out_specs=(
            pl.BlockSpec((_B, Cout, L), lambda n: (n, 0, 0)),
            pl.BlockSpec((1, Cout, 1), lambda n: (n, 0, 0)),
            pl.BlockSpec((1, Cout, 1), lambda n: (n, 0, 0)),
        ),
        scratch_shapes=[pltpu.VMEM((3 * Cin, _B * L + 2), jnp.bfloat16)],
        compiler_params=pltpu.CompilerParams(
            dimension_semantics=("parallel",),
            vmem_limit_bytes=_VMEM,
        ),
    )(xf, w)


def _finalize(s1_ref, s2_ref, g_ref, b_ref, m):
    mean = jnp.sum(s1_ref[...], axis=0) / m
    var = jnp.maximum(jnp.sum(s2_ref[...], axis=0) / m - mean * mean, 0.0)
    scale = g_ref[...] * jax.lax.rsqrt(var + _EPS)
    shift = b_ref[...] - mean * scale
    return scale, shift


def _bn_conv_stats_call(c1, w, s1, s2, g, b, *, N, H, W, Cin, Cout):
    L = H * W
    m = float(N * L)

    def body(x_ref, w_ref, s1_ref, s2_ref, g_ref, b_ref,
             o_ref, t1_ref, t2_ref, buf):
        scale, shift = _finalize(s1_ref, s2_ref, g_ref, b_ref, m)
        y = jnp.maximum(x_ref[...].astype(jnp.float32) * scale + shift, 0.0)
        _fill_shifted(buf, y.astype(jnp.bfloat16), Cin, L, W, _B)
        acc = _conv_taps(buf, w_ref, _B * L, W)
        t1_ref[0] = jnp.sum(acc, axis=1, keepdims=True)
        t2_ref[0] = jnp.sum(acc * acc, axis=1, keepdims=True)
        for b in range(_B):
            o_ref[b] = acc[:, b * L:(b + 1) * L].astype(jnp.bfloat16)

    return pl.pallas_call(
        body,
        out_shape=(
            jax.ShapeDtypeStruct((N, Cout, L), jnp.bfloat16),
            jax.ShapeDtypeStruct((N // _B, Cout, 1), jnp.float32),
            jax.ShapeDtypeStruct((N // _B, Cout, 1), jnp.float32),
        ),
        grid=(N // _B,),
        in_specs=[
            pl.BlockSpec((_B, Cin, L), lambda n: (n, 0, 0)),
            pl.BlockSpec((3, Cout, 3 * Cin), lambda n: (0, 0, 0)),
            pl.BlockSpec((N // _B, Cin, 1), lambda n: (0, 0, 0)),
            pl.BlockSpec((N // _B, Cin, 1), lambda n: (0, 0, 0)),
            pl.BlockSpec((Cin, 1), lambda n: (0, 0)),
            pl.BlockSpec((Cin, 1), lambda n: (0, 0)),
        ],
        out_specs=(
            pl.BlockSpec((_B, Cout, L), lambda n: (n, 0, 0)),
            pl.BlockSpec((1, Cout, 1), lambda n: (n, 0, 0)),
            pl.BlockSpec((1, Cout, 1), lambda n: (n, 0, 0)),
        ),
        scratch_shapes=[pltpu.VMEM((3 * Cin, _B * L + 2), jnp.bfloat16)],
        compiler_params=pltpu.CompilerParams(
            dimension_semantics=("parallel",),
            vmem_limit_bytes=_VMEM,
        ),
    )(c1, w, s1, s2, g, b)


def _bn_relu_out_call(c2, s1, s2, g, b, *, N, H, W, Cout):
    L = H * W
    m = float(N * L)

    def body(x_ref, s1_ref, s2_ref, g_ref, b_ref, o_ref):
        scale, shift = _finalize(s1_ref, s2_ref, g_ref, b_ref, m)
        o_ref[...] = jnp.maximum(
            x_ref[...].astype(jnp.float32) * scale + shift, 0.0)

    return pl.pallas_call(
        body,
        out_shape=jax.ShapeDtypeStruct((N, Cout, L), jnp.float32),
        grid=(N // _B,),
        in_specs=[
            pl.BlockSpec((_B, Cout, L), lambda n: (n, 0, 0)),
            pl.BlockSpec((N // _B, Cout, 1), lambda n: (0, 0, 0)),
            pl.BlockSpec((N // _B, Cout, 1), lambda n: (0, 0, 0)),
            pl.BlockSpec((Cout, 1), lambda n: (0, 0)),
            pl.BlockSpec((Cout, 1), lambda n: (0, 0)),
        ],
        out_specs=pl.BlockSpec((_B, Cout, L), lambda n: (n, 0, 0)),
        compiler_params=pltpu.CompilerParams(
            dimension_semantics=("parallel",),
            vmem_limit_bytes=_VMEM,
        ),
    )(c2, s1, s2, g, b)


def _pack_w(w, Cin, Cout):
    return (w.reshape(3, 3, Cin, Cout)
            .transpose(1, 3, 0, 2)
            .reshape(3, Cout, 3 * Cin)
            .astype(jnp.bfloat16))


def kernel(x, w1, g1, b1, w2, g2, b2):
    N, C0, H0, W0 = x.shape
    H, W = H0 // 2, W0 // 2
    L = H * W
    C1 = w1.shape[2]
    C2 = w2.shape[2]

    pooled = jnp.max(x.reshape(N, C0, H, 2, W, 2), axis=(3, 5))
    xf = pooled.astype(jnp.bfloat16).reshape(N, C0, L)

    w1p = _pack_w(w1, C0, C1)
    w2p = _pack_w(w2, C1, C2)
    g1r, b1r = g1.reshape(C1, 1), b1.reshape(C1, 1)
    g2r, b2r = g2.reshape(C2, 1), b2.reshape(C2, 1)

    c1, s1, s2 = _conv_stats_call(xf, w1p, N=N, H=H, W=W, Cin=C0, Cout=C1)
    c2, t1, t2 = _bn_conv_stats_call(c1, w2p, s1, s2, g1r, b1r,
                                     N=N, H=H, W=W, Cin=C1, Cout=C2)
    y = _bn_relu_out_call(c2, t1, t2, g2r, b2r, N=N, H=H, W=W, Cout=C2)
    return y.reshape(N, C2, H, W)


# speedup vs baseline: 9.5555x; 1.0023x over previous
import jax
import jax.numpy as jnp
from jax.experimental import pallas as pl
from jax.experimental.pallas import tpu as pltpu

_EPS = 1e-5
_VMEM = 64 * 1024 * 1024
_B = 4


def _edge_masks(L, W, dtype):
    col = jax.lax.broadcasted_iota(jnp.int32, (1, L), 1) % W
    left = (col != 0).astype(dtype)
    right = (col != W - 1).astype(dtype)
    return left, right


def _fill_shifted(buf, x, C, L, W, B):
    zero = jnp.zeros((3 * C, 2 * W), jnp.bfloat16)
    buf[:, 0:W + 1] = zero[:, :W + 1]
    for s in range(1, B):
        buf[:, pl.ds(s * L - W + 1, 2 * W)] = zero
    buf[:, pl.ds(B * L - W + 1, W + 1)] = zero[:, :W + 1]
    lo = {0: W + 1, 1: 1, 2: 0}
    hi = {0: L + 2, 1: L + 1, 2: L - W + 1}
    for b in range(B):
        for ky in range(3):
            i0 = max(lo[ky], 0 if b == 0 else 1)
            i1 = min(hi[ky], L + 2 if b == B - 1 else L + 1)
            s0 = i0 - 1 + (ky - 1) * W
            buf[ky * C:(ky + 1) * C, pl.ds(b * L + i0, i1 - i0)] = (
                x[b, :, s0:s0 + i1 - i0])


def _conv_taps(buf, w_ref, L, W):
    ml, mr = _edge_masks(L, W, jnp.bfloat16)
    acc = jnp.dot(w_ref[0], buf[:, pl.ds(0, L)] * ml,
                  preferred_element_type=jnp.float32)
    acc = acc + jnp.dot(w_ref[1], buf[:, pl.ds(1, L)],
                        preferred_element_type=jnp.float32)
    acc = acc + jnp.dot(w_ref[2], buf[:, pl.ds(2, L)] * mr,
                        preferred_element_type=jnp.float32)
    return acc


def _conv_stats_call(xf, w, *, N, H, W, Cin, Cout):
    L = H * W

    def body(x_ref, w_ref, o_ref, s1_ref, s2_ref, buf):
        _fill_shifted(buf, x_ref[...], Cin, L, W, _B)
        acc = _conv_taps(buf, w_ref, _B * L, W)
        s1_ref[0] = jnp.sum(acc, axis=1, keepdims=True)
        s2_ref[0] = jnp.sum(acc * acc, axis=1, keepdims=True)
        for b in range(_B):
            o_ref[b] = acc[:, b * L:(b + 1) * L].astype(jnp.bfloat16)

    return pl.pallas_call(
        body,
        out_shape=(
            jax.ShapeDtypeStruct((N, Cout, L), jnp.bfloat16),
            jax.ShapeDtypeStruct((N // _B, Cout, 1), jnp.float32),
            jax.ShapeDtypeStruct((N // _B, Cout, 1), jnp.float32),
        ),
        grid=(N // _B,),
        in_specs=[
            pl.BlockSpec((_B, Cin, L), lambda n: (n, 0, 0)),
            pl.BlockSpec((3, Cout, 3 * Cin), lambda n: (0, 0, 0)),
        ],
        out_specs=(
            pl.BlockSpec((_B, Cout, L), lambda n: (n, 0, 0)),
            pl.BlockSpec((1, Cout, 1), lambda n: (n, 0, 0)),
            pl.BlockSpec((1, Cout, 1), lambda n: (n, 0, 0)),
        ),
        scratch_shapes=[pltpu.VMEM((3 * Cin, _B * L + 2), jnp.bfloat16)],
        compiler_params=pltpu.CompilerParams(
            dimension_semantics=("parallel",),
            vmem_limit_bytes=_VMEM,
        ),
    )(xf, w)


def _finalize(s1_ref, s2_ref, g_ref, b_ref, m):
    mean = jnp.sum(s1_ref[...], axis=0) / m
    var = jnp.maximum(jnp.sum(s2_ref[...], axis=0) / m - mean * mean, 0.0)
    scale = g_ref[...] * jax.lax.rsqrt(var + _EPS)
    shift = b_ref[...] - mean * scale
    return scale, shift


def _bn_conv_stats_call(c1, w, s1, s2, g, b, *, N, H, W, Cin, Cout):
    L = H * W
    m = float(N * L)

    def body(x_ref, w_ref, s1_ref, s2_ref, g_ref, b_ref,
             o_ref, t1_ref, t2_ref, buf):
        scale, shift = _finalize(s1_ref, s2_ref, g_ref, b_ref, m)
        y = jnp.maximum(
            x_ref[...] * scale.astype(jnp.bfloat16) + shift.astype(jnp.bfloat16),
            jnp.asarray(0, jnp.bfloat16))
        _fill_shifted(buf, y, Cin, L, W, _B)
        acc = _conv_taps(buf, w_ref, _B * L, W)
        t1_ref[0] = jnp.sum(acc, axis=1, keepdims=True)
        t2_ref[0] = jnp.sum(acc * acc, axis=1, keepdims=True)
        for b in range(_B):
            o_ref[b] = acc[:, b * L:(b + 1) * L].astype(jnp.bfloat16)

    return pl.pallas_call(
        body,
        out_shape=(
            jax.ShapeDtypeStruct((N, Cout, L), jnp.bfloat16),
            jax.ShapeDtypeStruct((N // _B, Cout, 1), jnp.float32),
            jax.ShapeDtypeStruct((N // _B, Cout, 1), jnp.float32),
        ),
        grid=(N // _B,),
        in_specs=[
            pl.BlockSpec((_B, Cin, L), lambda n: (n, 0, 0)),
            pl.BlockSpec((3, Cout, 3 * Cin), lambda n: (0, 0, 0)),
            pl.BlockSpec((N // _B, Cin, 1), lambda n: (0, 0, 0)),
            pl.BlockSpec((N // _B, Cin, 1), lambda n: (0, 0, 0)),
            pl.BlockSpec((Cin, 1), lambda n: (0, 0)),
            pl.BlockSpec((Cin, 1), lambda n: (0, 0)),
        ],
        out_specs=(
            pl.BlockSpec((_B, Cout, L), lambda n: (n, 0, 0)),
            pl.BlockSpec((1, Cout, 1), lambda n: (n, 0, 0)),
            pl.BlockSpec((1, Cout, 1), lambda n: (n, 0, 0)),
        ),
        scratch_shapes=[pltpu.VMEM((3 * Cin, _B * L + 2), jnp.bfloat16)],
        compiler_params=pltpu.CompilerParams(
            dimension_semantics=("parallel",),
            vmem_limit_bytes=_VMEM,
        ),
    )(c1, w, s1, s2, g, b)


def _bn_relu_out_call(c2, s1, s2, g, b, *, N, H, W, Cout):
    L = H * W
    m = float(N * L)

    def body(x_ref, s1_ref, s2_ref, g_ref, b_ref, o_ref):
        scale, shift = _finalize(s1_ref, s2_ref, g_ref, b_ref, m)
        o_ref[...] = jnp.maximum(
            x_ref[...].astype(jnp.float32) * scale + shift, 0.0)

    return pl.pallas_call(
        body,
        out_shape=jax.ShapeDtypeStruct((N, Cout, L), jnp.float32),
        grid=(N // _B,),
        in_specs=[
            pl.BlockSpec((_B, Cout, L), lambda n: (n, 0, 0)),
            pl.BlockSpec((N // _B, Cout, 1), lambda n: (0, 0, 0)),
            pl.BlockSpec((N // _B, Cout, 1), lambda n: (0, 0, 0)),
            pl.BlockSpec((Cout, 1), lambda n: (0, 0)),
            pl.BlockSpec((Cout, 1), lambda n: (0, 0)),
        ],
        out_specs=pl.BlockSpec((_B, Cout, L), lambda n: (n, 0, 0)),
        compiler_params=pltpu.CompilerParams(
            dimension_semantics=("parallel",),
            vmem_limit_bytes=_VMEM,
        ),
    )(c2, s1, s2, g, b)


def _pack_w(w, Cin, Cout):
    return (w.reshape(3, 3, Cin, Cout)
            .transpose(1, 3, 0, 2)
            .reshape(3, Cout, 3 * Cin)
            .astype(jnp.bfloat16))


def kernel(x, w1, g1, b1, w2, g2, b2):
    N, C0, H0, W0 = x.shape
    H, W = H0 // 2, W0 // 2
    L = H * W
    C1 = w1.shape[2]
    C2 = w2.shape[2]

    pooled = jnp.max(x.reshape(N, C0, H, 2, W, 2), axis=(3, 5))
    xf = pooled.astype(jnp.bfloat16).reshape(N, C0, L)

    w1p = _pack_w(w1, C0, C1)
    w2p = _pack_w(w2, C1, C2)
    g1r, b1r = g1.reshape(C1, 1), b1.reshape(C1, 1)
    g2r, b2r = g2.reshape(C2, 1), b2.reshape(C2, 1)

    c1, s1, s2 = _conv_stats_call(xf, w1p, N=N, H=H, W=W, Cin=C0, Cout=C1)
    c2, t1, t2 = _bn_conv_stats_call(c1, w2p, s1, s2, g1r, b1r,
                                     N=N, H=H, W=W, Cin=C1, Cout=C2)
    y = _bn_relu_out_call(c2, t1, t2, g2r, b2r, N=N, H=H, W=W, Cout=C2)
    return y.reshape(N, C2, H, W)
